# baseline (speedup 1.0000x reference)
import functools

import jax
import jax.numpy as jnp
from jax.experimental import pallas as pl
from jax.experimental.pallas import tpu as pltpu
from jax.experimental.pallas import tpu_sc as plsc

DEPTH = 3
N_MOLS = 100

_NC = 2
_NS = 16
_NW = _NC * _NS
_GCHUNK = 80


def _sc_gather(table, idx):
    n = idx.shape[0]
    d = table.shape[1]
    per_w = n // _NW
    assert per_w * _NW == n and per_w % _GCHUNK == 0 and per_w % 8 == 0
    n_chunks = per_w // _GCHUNK
    mesh = plsc.VectorSubcoreMesh(core_axis_name="c", subcore_axis_name="s")

    @functools.partial(
        pl.kernel,
        mesh=mesh,
        out_type=jax.ShapeDtypeStruct((n, d), table.dtype),
        scratch_types=[
            pltpu.VMEM((_GCHUNK,), jnp.int32),
            pltpu.VMEM((_GCHUNK, d), table.dtype),
            pltpu.SemaphoreType.DMA,
        ],
    )
    def k(table_hbm, idx_hbm, out_hbm, idx_v, rows_v, sem):
        wid = jax.lax.axis_index("s") * _NC + jax.lax.axis_index("c")
        base = wid * per_w

        @pl.loop(0, n_chunks)
        def _(j):
            off = pl.multiple_of(base + j * _GCHUNK, 8)
            pltpu.sync_copy(idx_hbm.at[pl.ds(off, _GCHUNK)], idx_v)
            pltpu.async_copy(table_hbm.at[idx_v], rows_v, sem).wait()
            pltpu.sync_copy(rows_v, out_hbm.at[pl.ds(off, _GCHUNK)])

    return k(table, idx)


_BOND_BLK = 2000
_ATOM_BLK = 200


def _tc_init(f_bonds, w_i_t, w_h_t):
    n, fdim = f_bonds.shape
    h = w_i_t.shape[1]

    def body(fb, wi, wh, inp_ref, msgh_ref):
        inp = jnp.dot(fb[...], wi[...], preferred_element_type=jnp.float32)
        inp_ref[...] = inp
        msgh_ref[...] = jnp.dot(
            jnp.maximum(inp, 0.0), wh[...], preferred_element_type=jnp.float32
        )

    return pl.pallas_call(
        body,
        grid=(n // _BOND_BLK,),
        in_specs=[
            pl.BlockSpec((_BOND_BLK, fdim), lambda i: (i, 0)),
            pl.BlockSpec((fdim, h), lambda i: (0, 0)),
            pl.BlockSpec((h, h), lambda i: (0, 0)),
        ],
        out_specs=[
            pl.BlockSpec((_BOND_BLK, h), lambda i: (i, 0)),
            pl.BlockSpec((_BOND_BLK, h), lambda i: (i, 0)),
        ],
        out_shape=[
            jax.ShapeDtypeStruct((n, h), jnp.float32),
            jax.ShapeDtypeStruct((n, h), jnp.float32),
        ],
    )(f_bonds, w_i_t, w_h_t)


def _tc_segsum(nei):
    a, k, h = nei.shape

    def body(n_ref, o_ref):
        o_ref[...] = jnp.sum(n_ref[...], axis=1)

    return pl.pallas_call(
        body,
        grid=(a // _ATOM_BLK,),
        in_specs=[pl.BlockSpec((_ATOM_BLK, k, h), lambda i: (i, 0, 0))],
        out_specs=pl.BlockSpec((_ATOM_BLK, h), lambda i: (i, 0)),
        out_shape=jax.ShapeDtypeStruct((a, h), jnp.float32),
    )(nei)


def _tc_combine(inp, g1, rev, w_h_t):
    n, h = inp.shape
    matmul = w_h_t is not None

    def body(*refs):
        if matmul:
            inp_ref, g1_ref, rev_ref, wh_ref, o_ref = refs
        else:
            inp_ref, g1_ref, rev_ref, o_ref = refs
        m = jnp.maximum(inp_ref[...] + g1_ref[...] - rev_ref[...], 0.0)
        if matmul:
            m = jnp.dot(m, wh_ref[...], preferred_element_type=jnp.float32)
        o_ref[...] = m

    row_spec = pl.BlockSpec((_BOND_BLK, h), lambda i: (i, 0))
    in_specs = [row_spec, row_spec, row_spec]
    args = [inp, g1, rev]
    if matmul:
        in_specs.append(pl.BlockSpec((h, h), lambda i: (0, 0)))
        args.append(w_h_t)
    return pl.pallas_call(
        body,
        grid=(n // _BOND_BLK,),
        in_specs=in_specs,
        out_specs=row_spec,
        out_shape=jax.ShapeDtypeStruct((n, h), jnp.float32),
    )(*args)


def _tc_readout(f_atoms, nei, wo1_t, wo2_t, bias):
    a, fdim = f_atoms.shape
    _, k, h = nei.shape

    def body(fa_ref, n_ref, w1_ref, w2_ref, b_ref, o_ref):
        am = jnp.sum(n_ref[...], axis=1) * (1.0 / k)
        acc = jnp.dot(fa_ref[...], w1_ref[...], preferred_element_type=jnp.float32)
        acc += jnp.dot(am, w2_ref[...], preferred_element_type=jnp.float32)
        o_ref[...] = jnp.maximum(acc + b_ref[...], 0.0)

    return pl.pallas_call(
        body,
        grid=(a // _ATOM_BLK,),
        in_specs=[
            pl.BlockSpec((_ATOM_BLK, fdim), lambda i: (i, 0)),
            pl.BlockSpec((_ATOM_BLK, k, h), lambda i: (i, 0, 0)),
            pl.BlockSpec((fdim, h), lambda i: (0, 0)),
            pl.BlockSpec((h, h), lambda i: (0, 0)),
            pl.BlockSpec((1, h), lambda i: (0, 0)),
        ],
        out_specs=pl.BlockSpec((_ATOM_BLK, h), lambda i: (i, 0)),
        out_shape=jax.ShapeDtypeStruct((a, h), jnp.float32),
    )(f_atoms, nei, wo1_t, wo2_t, bias)


def kernel(f_atoms, f_bonds, a2b, b2a, b2revb, W_i, W_h, W_o_w, W_o_b):
    n_atoms, atom_fdim = f_atoms.shape
    max_nb = a2b.shape[1]
    h = W_i.shape[0]

    a2b_flat = a2b.reshape(-1).astype(jnp.int32)
    b2a = b2a.astype(jnp.int32)
    b2revb = b2revb.astype(jnp.int32)
    w_i_t = W_i.T
    w_h_t = W_h.T
    wo1_t = W_o_w[:, :atom_fdim].T
    wo2_t = W_o_w[:, atom_fdim:].T
    bias = W_o_b.reshape(1, h)

    inp, msgh = _tc_init(f_bonds, w_i_t, w_h_t)
    message = None
    for t in range(DEPTH - 1):
        nei = _sc_gather(msgh, a2b_flat)
        rev = _sc_gather(msgh, b2revb)
        amh = _tc_segsum(nei.reshape(n_atoms, max_nb, h))
        g1 = _sc_gather(amh, b2a)
        if t == DEPTH - 2:
            message = _tc_combine(inp, g1, rev, None)
        else:
            msgh = _tc_combine(inp, g1, rev, w_h_t)

    nei = _sc_gather(message, a2b_flat)
    out = _tc_readout(
        f_atoms, nei.reshape(n_atoms, max_nb, h), wo1_t, wo2_t, bias
    )
    return out.reshape(N_MOLS, n_atoms // N_MOLS, h)
